# baseline (device time: 82125 ns/iter reference)
import jax
import jax.numpy as jnp
from jax import lax
from jax.experimental import pallas as pl
from jax.experimental.pallas import tpu as pltpu

N_DEV = 4
SEG = 4


def kernel(t):
    m, n = t.shape
    ch = m // N_DEV
    hf = ch // 2
    sg = hf // SEG
    nh = N_DEV - 1

    def body(t_ref, out_ref, sb_cw, sb_ccw, rb_cw, rb_ccw,
             rs_cw_s, rs_cw_r, rs_ccw_s, rs_ccw_r,
             ag_cw_s, ag_cw_r, ag_ccw_s, ag_ccw_r):
        my = lax.axis_index("i")
        left = lax.rem(my + N_DEV - 1, N_DEV)
        right = lax.rem(my + 1, N_DEV)

        barrier_sem = pltpu.get_barrier_semaphore()
        for nbr in (left, right):
            pl.semaphore_signal(
                barrier_sem, inc=1,
                device_id=(nbr,), device_id_type=pl.DeviceIdType.MESH,
            )
        pl.semaphore_wait(barrier_sem, 2)

        def t_seg(idx, d, g):
            return t_ref[pl.ds(idx * ch + d * hf + g * sg, sg), :]

        def rs_rdma(d, h, g):
            sb, rb = (sb_cw, rb_cw) if d == 0 else (sb_ccw, rb_ccw)
            ss, rs_ = (rs_cw_s, rs_cw_r) if d == 0 else (rs_ccw_s, rs_ccw_r)
            tgt = right if d == 0 else left
            return pltpu.make_async_remote_copy(
                src_ref=sb.at[h, pl.ds(g * sg, sg), :],
                dst_ref=rb.at[h, pl.ds(g * sg, sg), :],
                send_sem=ss.at[h, g], recv_sem=rs_.at[h, g],
                device_id=(tgt,), device_id_type=pl.DeviceIdType.MESH,
            )

        def ag_rows(d, h, g):
            off = -h if d == 0 else h
            idx = lax.rem(my + 1 + off + N_DEV, N_DEV)
            return idx * ch + d * hf + g * sg

        def ag_rdma(d, h, g):
            ss, rs_ = (ag_cw_s, ag_cw_r) if d == 0 else (ag_ccw_s, ag_ccw_r)
            tgt = right if d == 0 else left
            row0 = ag_rows(d, h, g)
            return pltpu.make_async_remote_copy(
                src_ref=out_ref.at[pl.ds(row0, sg), :],
                dst_ref=out_ref.at[pl.ds(row0, sg), :],
                send_sem=ss.at[h, g], recv_sem=rs_.at[h, g],
                device_id=(tgt,), device_id_type=pl.DeviceIdType.MESH,
            )

        start_cw = my
        start_ccw = lax.rem(my + 2, N_DEV)
        for g in range(SEG):
            for d, idx0 in ((0, start_cw), (1, start_ccw)):
                sb = sb_cw if d == 0 else sb_ccw
                sb[0, pl.ds(g * sg, sg), :] = (
                    t_seg(idx0, d, g).astype(jnp.bfloat16))
                rs_rdma(d, h=0, g=g).start()

        own = lax.rem(my + 1, N_DEV)
        for h in range(nh):
            for g in range(SEG):
                for d in (0, 1):
                    rb = rb_cw if d == 0 else rb_ccw
                    idx = (lax.rem(my + N_DEV - 1 - h, N_DEV) if d == 0
                           else lax.rem(my + 3 + h, N_DEV))
                    rs_rdma(d, h, g).wait_recv()
                    acc = (rb[h, pl.ds(g * sg, sg), :].astype(jnp.float32)
                           + t_seg(idx, d, g))
                    if h < nh - 1:
                        sb = sb_cw if d == 0 else sb_ccw
                        sb[h + 1, pl.ds(g * sg, sg), :] = (
                            acc.astype(jnp.bfloat16))
                        rs_rdma(d, h + 1, g).start()
                    else:
                        r = jnp.maximum(acc, 0.0)
                        fval = jnp.tanh(acc) * acc * acc + r * r * r
                        row0 = own * ch + d * hf + g * sg
                        out_ref[pl.ds(row0, sg), :] = fval.astype(jnp.bfloat16)
                        ag_rdma(d, h=0, g=g).start()

        for h in range(nh):
            for g in range(SEG):
                for d in (0, 1):
                    ag_rdma(d, h, g).wait_recv()
                    if h < nh - 1:
                        ag_rdma(d, h + 1, g).start()

        for h in range(nh):
            for g in range(SEG):
                for d in (0, 1):
                    rs_rdma(d, h, g).wait_send()
                    ag_rdma(d, h, g).wait_send()

    return pl.pallas_call(
        body,
        out_shape=jax.ShapeDtypeStruct((m, n), jnp.bfloat16),
        in_specs=[pl.BlockSpec(memory_space=pltpu.VMEM)],
        out_specs=pl.BlockSpec(memory_space=pltpu.VMEM),
        scratch_shapes=[
            pltpu.VMEM((nh, hf, n), jnp.bfloat16),
            pltpu.VMEM((nh, hf, n), jnp.bfloat16),
            pltpu.VMEM((nh, hf, n), jnp.bfloat16),
            pltpu.VMEM((nh, hf, n), jnp.bfloat16),
            pltpu.SemaphoreType.DMA((nh, SEG)),
            pltpu.SemaphoreType.DMA((nh, SEG)),
            pltpu.SemaphoreType.DMA((nh, SEG)),
            pltpu.SemaphoreType.DMA((nh, SEG)),
            pltpu.SemaphoreType.DMA((nh, SEG)),
            pltpu.SemaphoreType.DMA((nh, SEG)),
            pltpu.SemaphoreType.DMA((nh, SEG)),
            pltpu.SemaphoreType.DMA((nh, SEG)),
        ],
        compiler_params=pltpu.CompilerParams(collective_id=0),
    )(t)


# device time: 77139 ns/iter; 1.0646x vs baseline; 1.0646x over previous
import jax
import jax.numpy as jnp
from jax import lax
from jax.experimental import pallas as pl
from jax.experimental.pallas import tpu as pltpu

N_DEV = 4
SEG = 2


def kernel(t):
    m, n = t.shape
    ch = m // N_DEV
    hf = ch // 2
    sg = hf // SEG
    nh = N_DEV - 1

    def body(t_ref, out_ref, sb_cw, sb_ccw, rb_cw, rb_ccw, tv,
             rs_cw_s, rs_cw_r, rs_ccw_s, rs_ccw_r,
             ag_cw_s, ag_cw_r, ag_ccw_s, ag_ccw_r, ld_sems):
        my = lax.axis_index("i")
        left = lax.rem(my + N_DEV - 1, N_DEV)
        right = lax.rem(my + 1, N_DEV)

        def load_half(idx, d, sem_i):
            rows = pl.ds(idx * ch + d * hf, hf)
            return pltpu.make_async_copy(
                t_ref.at[rows, :], tv.at[rows, :], ld_sems.at[sem_i])

        start_cw = my
        start_ccw = lax.rem(my + 2, N_DEV)
        load_half(start_cw, 0, 0).start()
        load_half(start_ccw, 1, 1).start()

        barrier_sem = pltpu.get_barrier_semaphore()
        for nbr in (left, right):
            pl.semaphore_signal(
                barrier_sem, inc=1,
                device_id=(nbr,), device_id_type=pl.DeviceIdType.MESH,
            )
        pl.semaphore_wait(barrier_sem, 2)

        def t_seg(idx, d, g):
            return tv[pl.ds(idx * ch + d * hf + g * sg, sg), :]

        def rs_rdma(d, h, g):
            sb, rb = (sb_cw, rb_cw) if d == 0 else (sb_ccw, rb_ccw)
            ss, rs_ = (rs_cw_s, rs_cw_r) if d == 0 else (rs_ccw_s, rs_ccw_r)
            tgt = right if d == 0 else left
            return pltpu.make_async_remote_copy(
                src_ref=sb.at[h, pl.ds(g * sg, sg), :],
                dst_ref=rb.at[h, pl.ds(g * sg, sg), :],
                send_sem=ss.at[h, g], recv_sem=rs_.at[h, g],
                device_id=(tgt,), device_id_type=pl.DeviceIdType.MESH,
            )

        def ag_rows(d, h, g):
            off = -h if d == 0 else h
            idx = lax.rem(my + 1 + off + N_DEV, N_DEV)
            return idx * ch + d * hf + g * sg

        def ag_rdma(d, h, g):
            ss, rs_ = (ag_cw_s, ag_cw_r) if d == 0 else (ag_ccw_s, ag_ccw_r)
            tgt = right if d == 0 else left
            row0 = ag_rows(d, h, g)
            return pltpu.make_async_remote_copy(
                src_ref=out_ref.at[pl.ds(row0, sg), :],
                dst_ref=out_ref.at[pl.ds(row0, sg), :],
                send_sem=ss.at[h, g], recv_sem=rs_.at[h, g],
                device_id=(tgt,), device_id_type=pl.DeviceIdType.MESH,
            )

        load_half(start_cw, 0, 0).wait()
        load_half(start_ccw, 1, 1).wait()
        for g in range(SEG):
            for d, idx0 in ((0, start_cw), (1, start_ccw)):
                sb = sb_cw if d == 0 else sb_ccw
                sb[0, pl.ds(g * sg, sg), :] = (
                    t_seg(idx0, d, g).astype(jnp.bfloat16))
                rs_rdma(d, h=0, g=g).start()

        for h in range(nh):
            for d in (0, 1):
                idx = (lax.rem(my + N_DEV - 1 - h, N_DEV) if d == 0
                       else lax.rem(my + 3 + h, N_DEV))
                load_half(idx, d, 2 + 2 * h + d).start()

        own = lax.rem(my + 1, N_DEV)
        for h in range(nh):
            for g in range(SEG):
                for d in (0, 1):
                    rb = rb_cw if d == 0 else rb_ccw
                    idx = (lax.rem(my + N_DEV - 1 - h, N_DEV) if d == 0
                           else lax.rem(my + 3 + h, N_DEV))
                    if g == 0:
                        load_half(idx, d, 2 + 2 * h + d).wait()
                    rs_rdma(d, h, g).wait_recv()
                    acc = (rb[h, pl.ds(g * sg, sg), :].astype(jnp.float32)
                           + t_seg(idx, d, g))
                    if h < nh - 1:
                        sb = sb_cw if d == 0 else sb_ccw
                        sb[h + 1, pl.ds(g * sg, sg), :] = (
                            acc.astype(jnp.bfloat16))
                        rs_rdma(d, h + 1, g).start()
                    else:
                        r = jnp.maximum(acc, 0.0)
                        fval = jnp.tanh(acc) * acc * acc + r * r * r
                        row0 = own * ch + d * hf + g * sg
                        out_ref[pl.ds(row0, sg), :] = fval.astype(jnp.bfloat16)
                        ag_rdma(d, h=0, g=g).start()

        for h in range(nh):
            for g in range(SEG):
                for d in (0, 1):
                    ag_rdma(d, h, g).wait_recv()
                    if h < nh - 1:
                        ag_rdma(d, h + 1, g).start()

        for h in range(nh):
            for g in range(SEG):
                for d in (0, 1):
                    rs_rdma(d, h, g).wait_send()
                    ag_rdma(d, h, g).wait_send()

    return pl.pallas_call(
        body,
        out_shape=jax.ShapeDtypeStruct((m, n), jnp.bfloat16),
        in_specs=[pl.BlockSpec(memory_space=pl.ANY)],
        out_specs=pl.BlockSpec(memory_space=pltpu.VMEM),
        scratch_shapes=[
            pltpu.VMEM((nh, hf, n), jnp.bfloat16),
            pltpu.VMEM((nh, hf, n), jnp.bfloat16),
            pltpu.VMEM((nh, hf, n), jnp.bfloat16),
            pltpu.VMEM((nh, hf, n), jnp.bfloat16),
            pltpu.VMEM((m, n), jnp.float32),
            pltpu.SemaphoreType.DMA((nh, SEG)),
            pltpu.SemaphoreType.DMA((nh, SEG)),
            pltpu.SemaphoreType.DMA((nh, SEG)),
            pltpu.SemaphoreType.DMA((nh, SEG)),
            pltpu.SemaphoreType.DMA((nh, SEG)),
            pltpu.SemaphoreType.DMA((nh, SEG)),
            pltpu.SemaphoreType.DMA((nh, SEG)),
            pltpu.SemaphoreType.DMA((nh, SEG)),
            pltpu.SemaphoreType.DMA((2 + 2 * nh,)),
        ],
        compiler_params=pltpu.CompilerParams(collective_id=0),
    )(t)


# device time: 77095 ns/iter; 1.0652x vs baseline; 1.0006x over previous
import jax
import jax.numpy as jnp
from jax import lax
from jax.experimental import pallas as pl
from jax.experimental.pallas import tpu as pltpu

N_DEV = 4
SEG = 2


def kernel(t):
    m, n = t.shape
    ch = m // N_DEV
    hf = ch // 2
    sg = hf // SEG
    nh = N_DEV - 1

    def body(t_ref, out_ref, sb_cw, sb_ccw, rb_cw, rb_ccw, tv, fstage,
             rs_cw_s, rs_cw_r, rs_ccw_s, rs_ccw_r,
             ag_cw_s, ag_cw_r, ag_ccw_s, ag_ccw_r, ld_sems, st_sem):
        my = lax.axis_index("i")
        left = lax.rem(my + N_DEV - 1, N_DEV)
        right = lax.rem(my + 1, N_DEV)

        def load_half(idx, d, sem_i):
            rows = pl.ds(idx * ch + d * hf, hf)
            return pltpu.make_async_copy(
                t_ref.at[rows, :], tv.at[rows, :], ld_sems.at[sem_i])

        start_cw = my
        start_ccw = lax.rem(my + 2, N_DEV)
        load_half(start_cw, 0, 0).start()
        load_half(start_ccw, 1, 1).start()

        barrier_sem = pltpu.get_barrier_semaphore()
        for nbr in (left, right):
            pl.semaphore_signal(
                barrier_sem, inc=1,
                device_id=(nbr,), device_id_type=pl.DeviceIdType.MESH,
            )
        pl.semaphore_wait(barrier_sem, 2)

        def t_seg(idx, d, g):
            return tv[pl.ds(idx * ch + d * hf + g * sg, sg), :]

        def rs_rdma(d, h, g):
            sb, rb = (sb_cw, rb_cw) if d == 0 else (sb_ccw, rb_ccw)
            ss, rs_ = (rs_cw_s, rs_cw_r) if d == 0 else (rs_ccw_s, rs_ccw_r)
            tgt = right if d == 0 else left
            return pltpu.make_async_remote_copy(
                src_ref=sb.at[h, pl.ds(g * sg, sg), :],
                dst_ref=rb.at[h, pl.ds(g * sg, sg), :],
                send_sem=ss.at[h, g], recv_sem=rs_.at[h, g],
                device_id=(tgt,), device_id_type=pl.DeviceIdType.MESH,
            )

        def ag_rows(d, h, g):
            off = -h if d == 0 else h
            idx = lax.rem(my + 1 + off + N_DEV, N_DEV)
            return idx * ch + d * hf + g * sg

        def ag_rdma(d, h, g):
            ss, rs_ = (ag_cw_s, ag_cw_r) if d == 0 else (ag_ccw_s, ag_ccw_r)
            tgt = right if d == 0 else left
            row0 = ag_rows(d, h, g)
            if h == 0:
                src = fstage.at[pl.ds(d * hf + g * sg, sg), :]
            else:
                src = out_ref.at[pl.ds(row0, sg), :]
            return pltpu.make_async_remote_copy(
                src_ref=src,
                dst_ref=out_ref.at[pl.ds(row0, sg), :],
                send_sem=ss.at[h, g], recv_sem=rs_.at[h, g],
                device_id=(tgt,), device_id_type=pl.DeviceIdType.MESH,
            )

        load_half(start_cw, 0, 0).wait()
        load_half(start_ccw, 1, 1).wait()
        for g in range(SEG):
            for d, idx0 in ((0, start_cw), (1, start_ccw)):
                sb = sb_cw if d == 0 else sb_ccw
                sb[0, pl.ds(g * sg, sg), :] = (
                    t_seg(idx0, d, g).astype(jnp.bfloat16))
                rs_rdma(d, h=0, g=g).start()

        for h in range(nh):
            for d in (0, 1):
                idx = (lax.rem(my + N_DEV - 1 - h, N_DEV) if d == 0
                       else lax.rem(my + 3 + h, N_DEV))
                load_half(idx, d, 2 + 2 * h + d).start()

        own = lax.rem(my + 1, N_DEV)
        for h in range(nh):
            for g in range(SEG):
                for d in (0, 1):
                    rb = rb_cw if d == 0 else rb_ccw
                    idx = (lax.rem(my + N_DEV - 1 - h, N_DEV) if d == 0
                           else lax.rem(my + 3 + h, N_DEV))
                    if g == 0:
                        load_half(idx, d, 2 + 2 * h + d).wait()
                    rs_rdma(d, h, g).wait_recv()
                    acc = (rb[h, pl.ds(g * sg, sg), :].astype(jnp.float32)
                           + t_seg(idx, d, g))
                    if h < nh - 1:
                        sb = sb_cw if d == 0 else sb_ccw
                        sb[h + 1, pl.ds(g * sg, sg), :] = (
                            acc.astype(jnp.bfloat16))
                        rs_rdma(d, h + 1, g).start()
                    else:
                        r = jnp.maximum(acc, 0.0)
                        fval = jnp.tanh(acc) * acc * acc + r * r * r
                        fstage[pl.ds(d * hf + g * sg, sg), :] = (
                            fval.astype(jnp.bfloat16))
                        ag_rdma(d, h=0, g=g).start()

        own_store = pltpu.make_async_copy(
            fstage, out_ref.at[pl.ds(own * ch, ch), :], st_sem)
        own_store.start()

        for h in range(nh):
            for g in range(SEG):
                for d in (0, 1):
                    ag_rdma(d, h, g).wait_recv()
                    if h < nh - 1:
                        ag_rdma(d, h + 1, g).start()

        own_store.wait()
        for h in range(nh):
            for g in range(SEG):
                for d in (0, 1):
                    rs_rdma(d, h, g).wait_send()
                    ag_rdma(d, h, g).wait_send()

    return pl.pallas_call(
        body,
        out_shape=jax.ShapeDtypeStruct((m, n), jnp.bfloat16),
        in_specs=[pl.BlockSpec(memory_space=pl.ANY)],
        out_specs=pl.BlockSpec(memory_space=pl.ANY),
        scratch_shapes=[
            pltpu.VMEM((nh, hf, n), jnp.bfloat16),
            pltpu.VMEM((nh, hf, n), jnp.bfloat16),
            pltpu.VMEM((nh, hf, n), jnp.bfloat16),
            pltpu.VMEM((nh, hf, n), jnp.bfloat16),
            pltpu.VMEM((m, n), jnp.float32),
            pltpu.VMEM((ch, n), jnp.bfloat16),
            pltpu.SemaphoreType.DMA((nh, SEG)),
            pltpu.SemaphoreType.DMA((nh, SEG)),
            pltpu.SemaphoreType.DMA((nh, SEG)),
            pltpu.SemaphoreType.DMA((nh, SEG)),
            pltpu.SemaphoreType.DMA((nh, SEG)),
            pltpu.SemaphoreType.DMA((nh, SEG)),
            pltpu.SemaphoreType.DMA((nh, SEG)),
            pltpu.SemaphoreType.DMA((nh, SEG)),
            pltpu.SemaphoreType.DMA((2 + 2 * nh,)),
            pltpu.SemaphoreType.DMA,
        ],
        compiler_params=pltpu.CompilerParams(collective_id=0),
    )(t)
